# baseline (speedup 1.0000x reference)
import functools

import jax
import jax.numpy as jnp
import numpy as np
from jax import lax
from jax.experimental import pallas as pl
from jax.experimental.pallas import tpu as pltpu
from jax.experimental.pallas import tpu_sc as plsc

N = 10000
E = 320000
TW = 25
H = 128
L_PDE = 16.0
TMAX = 4.0
DT = 0.01
EPS = 1e-5

NC = 2
NS = 16
NW = NC * NS
CHUNK = 128
EPAD = 327680
EPW = EPAD // NW
CPW = EPW // CHUNK
NPAD = 10240
NPT = NPAD // NS
DUMMY = NPAD - 8

_f32 = jnp.float32


def _swish(x):
    return x * (1.0 / (1.0 + jnp.exp(-x)))


def _tc_embed(ucat, w1, b1, w2, b2):
    blk = 1000

    def body(u_ref, w1_ref, b1_ref, w2_ref, b2_ref, h_ref):
        x = jnp.dot(u_ref[...], w1_ref[...], preferred_element_type=_f32)
        x = _swish(x + b1_ref[...])
        y = jnp.dot(x, w2_ref[...], preferred_element_type=_f32)
        h_ref[...] = _swish(y + b2_ref[...])

    return pl.pallas_call(
        body,
        grid=(N // blk,),
        in_specs=[
            pl.BlockSpec((blk, 32), lambda i: (i, 0)),
            pl.BlockSpec((32, H), lambda i: (0, 0)),
            pl.BlockSpec((1, H), lambda i: (0, 0)),
            pl.BlockSpec((H, H), lambda i: (0, 0)),
            pl.BlockSpec((1, H), lambda i: (0, 0)),
        ],
        out_specs=pl.BlockSpec((blk, H), lambda i: (i, 0)),
        out_shape=jax.ShapeDtypeStruct((N, H), _f32),
    )(ucat, w1, b1, w2, b2)


def _tc_pre(h, ucat, wa, wb, wcd, wcs, m1b, ua, ucu, u1b):
    blk = 1000

    def body(h_ref, u_ref, wa_ref, wb_ref, wcd_ref, wcs_ref, m1b_ref,
             ua_ref, ucu_ref, u1b_ref, a_ref, b_ref, p_ref):
        hv = h_ref[...]
        uv = u_ref[...]
        a_ref[...] = (jnp.dot(hv, wa_ref[...], preferred_element_type=_f32)
                      + jnp.dot(uv, wcd_ref[...], preferred_element_type=_f32)
                      + m1b_ref[...])
        b_ref[...] = (jnp.dot(hv, wb_ref[...], preferred_element_type=_f32)
                      + jnp.dot(uv, wcs_ref[...], preferred_element_type=_f32))
        p_ref[...] = (jnp.dot(hv, ua_ref[...], preferred_element_type=_f32)
                      + jnp.dot(uv, ucu_ref[...], preferred_element_type=_f32)
                      + u1b_ref[...])

    wspec = lambda shape: pl.BlockSpec(shape, lambda i: (0, 0))
    return pl.pallas_call(
        body,
        grid=(N // blk,),
        in_specs=[
            pl.BlockSpec((blk, H), lambda i: (i, 0)),
            pl.BlockSpec((blk, 32), lambda i: (i, 0)),
            wspec((H, H)), wspec((H, H)), wspec((32, H)), wspec((32, H)),
            wspec((1, H)), wspec((H, H)), wspec((32, H)), wspec((1, H)),
        ],
        out_specs=[pl.BlockSpec((blk, H), lambda i: (i, 0))] * 3,
        out_shape=[jax.ShapeDtypeStruct((N, H), _f32)] * 3,
    )(h, ucat, wa, wb, wcd, wcs, m1b, ua, ucu, u1b)


def _tc_edge(ad, bs, m2w, m2b):
    blk = 2048

    def body(a_ref, b_ref, w_ref, bias_ref, m_ref):
        t = _swish(a_ref[...] + b_ref[...])
        y = jnp.dot(t, w_ref[...], preferred_element_type=_f32) + bias_ref[...]
        m_ref[...] = _swish(y)

    return pl.pallas_call(
        body,
        grid=(EPAD // blk,),
        in_specs=[
            pl.BlockSpec((blk, H), lambda i: (i, 0)),
            pl.BlockSpec((blk, H), lambda i: (i, 0)),
            pl.BlockSpec((H, H), lambda i: (0, 0)),
            pl.BlockSpec((1, H), lambda i: (0, 0)),
        ],
        out_specs=pl.BlockSpec((blk, H), lambda i: (i, 0)),
        out_shape=jax.ShapeDtypeStruct((EPAD, H), _f32),
    )(ad, bs, m2w, m2b)


def _tc_upd(h, preu, p_parts, degp, ub, u2w, u2b):
    def body(h_ref, pre_ref, p_ref, dg_ref, ub_ref, u2w_ref, u2b_ref, o_ref):
        deg = jnp.maximum(dg_ref[0, :N] + dg_ref[1, :N], 1.0)
        agg = (p_ref[0, :N] + p_ref[1, :N]) * (1.0 / deg)
        x = pre_ref[...] + jnp.dot(agg, ub_ref[...], preferred_element_type=_f32)
        x = _swish(x)
        y = jnp.dot(x, u2w_ref[...], preferred_element_type=_f32) + u2b_ref[...]
        hn = h_ref[...] + _swish(y)
        mean = jnp.mean(hn, axis=0, keepdims=True)
        d = hn - mean
        var = jnp.mean(d * d, axis=0, keepdims=True)
        o_ref[...] = d / jnp.sqrt(var + EPS)

    return pl.pallas_call(
        body,
        out_shape=jax.ShapeDtypeStruct((N, H), _f32),
    )(h, preu, p_parts, degp, ub, u2w, u2b)


def _tc_decoder(h, m1f, c1b, m2f, b2row, u):
    blk = 1000

    def body(h_ref, m1_ref, c1_ref, m2_ref, b2_ref, u_ref, o_ref):
        z = jnp.dot(h_ref[...], m1_ref[...], preferred_element_type=_f32)
        z = _swish(z + c1_ref[...])
        diff = jnp.dot(z, m2_ref[...], preferred_element_type=_f32) + b2_ref[...]
        dtv = DT * (lax.broadcasted_iota(jnp.int32, (blk, TW), 1)
                    .astype(_f32) + 1.0)
        o_ref[...] = u_ref[:, TW - 1:TW] + dtv * diff

    return pl.pallas_call(
        body,
        grid=(N // blk,),
        in_specs=[
            pl.BlockSpec((blk, H), lambda i: (i, 0)),
            pl.BlockSpec((H, 304), lambda i: (0, 0)),
            pl.BlockSpec((1, 304), lambda i: (0, 0)),
            pl.BlockSpec((304, TW), lambda i: (0, 0)),
            pl.BlockSpec((1, TW), lambda i: (0, 0)),
            pl.BlockSpec((blk, TW), lambda i: (i, 0)),
        ],
        out_specs=pl.BlockSpec((blk, TW), lambda i: (i, 0)),
        out_shape=jax.ShapeDtypeStruct((N, TW), _f32),
    )(h, m1f, c1b, m2f, b2row, u)


def _sc_gather(a, b, dstg3, srcg3):
    mesh = plsc.VectorSubcoreMesh(core_axis_name="c", subcore_axis_name="s")

    @functools.partial(
        pl.kernel,
        mesh=mesh,
        out_type=[jax.ShapeDtypeStruct((EPAD, H), _f32)] * 2,
        scratch_types=[
            pltpu.VMEM((CPW, CHUNK), jnp.int32),
            pltpu.VMEM((CPW, CHUNK), jnp.int32),
            pltpu.VMEM((2, CHUNK, H), _f32),
            pltpu.VMEM((2, CHUNK, H), _f32),
            pltpu.SemaphoreType.DMA,
            pltpu.SemaphoreType.DMA,
        ],
    )
    def k(a_h, b_h, d3_h, s3_h, ad_h, bs_h, idx_d, idx_s, buf_a, buf_b, s0, s1):
        c = lax.axis_index("c")
        s = lax.axis_index("s")
        w = s * NC + c
        pltpu.sync_copy(d3_h.at[w], idx_d)
        pltpu.sync_copy(s3_h.at[w], idx_s)
        base = w * EPW
        sems = (s0, s1)

        def issue(g, slot):
            pltpu.async_copy(a_h.at[idx_d.at[g]], buf_a.at[slot], sems[slot])
            pltpu.async_copy(b_h.at[idx_s.at[g]], buf_b.at[slot], sems[slot])

        def drain(slot, cur):
            pltpu.make_async_copy(a_h.at[idx_d.at[0]], buf_a.at[slot],
                                  sems[slot]).wait()
            pltpu.make_async_copy(b_h.at[idx_s.at[0]], buf_b.at[slot],
                                  sems[slot]).wait()
            pltpu.sync_copy(buf_a.at[slot],
                            ad_h.at[pl.ds(base + cur * CHUNK, CHUNK)])
            pltpu.sync_copy(buf_b.at[slot],
                            bs_h.at[pl.ds(base + cur * CHUNK, CHUNK)])

        issue(0, 0)
        issue(1, 1)

        @pl.loop(0, CPW, step=2)
        def _(g):
            for bslot in range(2):
                cur = g + bslot
                drain(bslot, cur)

                @pl.when(cur + 2 < CPW)
                def _():
                    issue(cur + 2, bslot)

    return k(a, b, dstg3, srcg3)


def _sc_scatter(m, dst3, zeros_nd):
    mesh = plsc.VectorSubcoreMesh(core_axis_name="c", subcore_axis_name="s")

    @functools.partial(
        pl.kernel,
        mesh=mesh,
        out_type=jax.ShapeDtypeStruct((2, NPAD, H), _f32),
        scratch_types=[
            pltpu.VMEM((CPW, CHUNK), jnp.int32),
            pltpu.VMEM((2, CHUNK, H), _f32),
            pltpu.VMEM_SHARED((NPAD, H), _f32),
            pltpu.SemaphoreType.DMA,
            pltpu.SemaphoreType.DMA,
        ],
    )
    def k(m_h, d3_h, z_h, out_h, idx_d, mbuf, acc, s0, s1):
        c = lax.axis_index("c")
        s = lax.axis_index("s")
        w = s * NC + c
        pltpu.sync_copy(d3_h.at[w], idx_d)
        pltpu.sync_copy(z_h.at[pl.ds(s * NPT, NPT)], acc.at[pl.ds(s * NPT, NPT)])
        plsc.subcore_barrier()
        base = w * EPW
        sems = (s0, s1)

        def load(g, slot):
            pltpu.async_copy(m_h.at[pl.ds(base + g * CHUNK, CHUNK)],
                             mbuf.at[slot], sems[slot])

        def drain(slot, cur):
            pltpu.make_async_copy(m_h.at[pl.ds(0, CHUNK)],
                                  mbuf.at[slot], sems[slot]).wait()
            pltpu.sync_copy(mbuf.at[slot], acc.at[idx_d.at[cur]], add=True)

        load(0, 0)
        load(1, 1)

        @pl.loop(0, CPW, step=2)
        def _(g):
            for bslot in range(2):
                cur = g + bslot
                drain(bslot, cur)

                @pl.when(cur + 2 < CPW)
                def _():
                    load(cur + 2, bslot)

        plsc.subcore_barrier()
        pltpu.sync_copy(acc.at[pl.ds(s * NPT, NPT)],
                        out_h.at[c, pl.ds(s * NPT, NPT)])

    return k(m, dst3, zeros_nd)


def _sc_degree(dst3, zeros_nw, ones_cw):
    mesh = plsc.VectorSubcoreMesh(core_axis_name="c", subcore_axis_name="s")
    DW = 128

    @functools.partial(
        pl.kernel,
        mesh=mesh,
        out_type=jax.ShapeDtypeStruct((2, NPAD, DW), _f32),
        scratch_types=[
            pltpu.VMEM((CPW, CHUNK), jnp.int32),
            pltpu.VMEM((CHUNK, DW), _f32),
            pltpu.VMEM_SHARED((NPAD, DW), _f32),
        ],
    )
    def k(d3_h, z_h, o_h, out_h, idx_d, ones_v, acc):
        c = lax.axis_index("c")
        s = lax.axis_index("s")
        w = s * NC + c
        pltpu.sync_copy(d3_h.at[w], idx_d)
        pltpu.sync_copy(o_h, ones_v)
        pltpu.sync_copy(z_h.at[pl.ds(s * NPT, NPT)], acc.at[pl.ds(s * NPT, NPT)])
        plsc.subcore_barrier()

        @pl.loop(0, CPW)
        def _(g):
            pltpu.sync_copy(ones_v, acc.at[idx_d.at[g]], add=True)

        plsc.subcore_barrier()
        pltpu.sync_copy(acc.at[pl.ds(s * NPT, NPT)],
                        out_h.at[c, pl.ds(s * NPT, NPT)])

    return k(dst3, zeros_nw, ones_cw)


def kernel(u, pos, edge_index, batch, params):
    del batch
    pos_x = pos[:, 1:2] / L_PDE
    variables = pos[:, 0:1] / TMAX
    ucat = jnp.concatenate(
        [u, pos_x, variables, jnp.zeros((N, 5), _f32)], axis=1)

    w1p = jnp.concatenate(
        [params['emb_W1'], jnp.zeros((5, H), _f32)], axis=0)
    h = _tc_embed(ucat, w1p, params['emb_b1'][None, :],
                  params['emb_W2'], params['emb_b2'][None, :])

    src = edge_index[0]
    dst = edge_index[1]
    npad_e = EPAD - E
    dstg3 = jnp.concatenate(
        [dst, jnp.zeros((npad_e,), jnp.int32)]).reshape(NW, CPW, CHUNK)
    srcg3 = jnp.concatenate(
        [src, jnp.zeros((npad_e,), jnp.int32)]).reshape(NW, CPW, CHUNK)
    dsts3 = jnp.concatenate(
        [dst, jnp.full((npad_e,), DUMMY, jnp.int32)]).reshape(NW, CPW, CHUNK)

    zeros_nd = jnp.zeros((NPAD, H), _f32)
    zeros_nw = jnp.zeros((NPAD, 128), _f32)
    ones_cw = jnp.ones((CHUNK, 128), _f32)
    degp = _sc_degree(dsts3, zeros_nw, ones_cw)[:, :, 0:1]

    pad5 = jnp.zeros((5, H), _f32)
    pad1 = jnp.zeros((1, H), _f32)
    for lp in params['layers']:
        w = lp['m1W']
        wa = w[0:H]
        wb = w[H:2 * H]
        wc = w[2 * H:2 * H + TW]
        wd = w[2 * H + TW:2 * H + TW + 1]
        we = w[2 * H + TW + 1:2 * H + TW + 2]
        wcd = jnp.concatenate([wc, wd, we, pad5], axis=0)
        wcs = jnp.concatenate([-wc, -wd, pad1, pad5], axis=0)
        uw = lp['u1W']
        ua = uw[0:H]
        ub = uw[H:2 * H]
        uc = uw[2 * H:2 * H + 1]
        ucu = jnp.concatenate([jnp.zeros((TW + 1, H), _f32), uc, pad5], axis=0)

        a, b, preu = _tc_pre(h, ucat, wa, wb, wcd, wcs, lp['m1b'][None, :],
                             ua, ucu, lp['u1b'][None, :])
        ad, bs = _sc_gather(a, b, dstg3, srcg3)
        m = _tc_edge(ad, bs, lp['m2W'], lp['m2b'][None, :])
        p_parts = _sc_scatter(m, dsts3, zeros_nd)
        h = _tc_upd(h, preu, p_parts, degp, ub, lp['u2W'],
                    lp['u2b'][None, :])

    P1 = (H - 16) // 3 + 1
    s1 = np.zeros((P1, H, 16), np.float32)
    for p in range(P1):
        for kk in range(16):
            s1[p, 3 * p + kk, kk] = 1.0
    s2 = np.zeros((TW, P1, 14), np.float32)
    for q in range(TW):
        for kk in range(14):
            s2[q, q + kk, kk] = 1.0
    m1f = jnp.einsum('pjk,ck->jpc', jnp.asarray(s1),
                     params['conv_W1'][:, 0, :]).reshape(H, P1 * 8)
    m2f = jnp.einsum('qpk,ck->pcq', jnp.asarray(s2),
                     params['conv_W2'][0]).reshape(P1 * 8, TW)
    c1b = jnp.tile(params['conv_b1'], P1)[None, :]
    b2row = jnp.broadcast_to(params['conv_b2'], (TW,))[None, :]
    return _tc_decoder(h, m1f, c1b, m2f, b2row, u)


# speedup vs baseline: 1.0217x; 1.0217x over previous
import functools

import jax
import jax.numpy as jnp
import numpy as np
from jax import lax
from jax.experimental import pallas as pl
from jax.experimental.pallas import tpu as pltpu
from jax.experimental.pallas import tpu_sc as plsc

N = 10000
E = 320000
TW = 25
H = 128
L_PDE = 16.0
TMAX = 4.0
DT = 0.01
EPS = 1e-5

NC = 2
NS = 16
NW = NC * NS
CHUNK = 80
EPAD = 327680
EPW = EPAD // NW
CPW = EPW // CHUNK
NPAD = 10240
NPT = NPAD // NS
DUMMY = NPAD - 8

_f32 = jnp.float32


def _swish(x):
    return x * (1.0 / (1.0 + jnp.exp(-x)))


def _tc_embed(ucat, w1, b1, w2, b2):
    blk = 1000

    def body(u_ref, w1_ref, b1_ref, w2_ref, b2_ref, h_ref):
        x = jnp.dot(u_ref[...], w1_ref[...], preferred_element_type=_f32)
        x = _swish(x + b1_ref[...])
        y = jnp.dot(x, w2_ref[...], preferred_element_type=_f32)
        h_ref[...] = _swish(y + b2_ref[...])

    return pl.pallas_call(
        body,
        grid=(N // blk,),
        in_specs=[
            pl.BlockSpec((blk, 32), lambda i: (i, 0)),
            pl.BlockSpec((32, H), lambda i: (0, 0)),
            pl.BlockSpec((1, H), lambda i: (0, 0)),
            pl.BlockSpec((H, H), lambda i: (0, 0)),
            pl.BlockSpec((1, H), lambda i: (0, 0)),
        ],
        out_specs=pl.BlockSpec((blk, H), lambda i: (i, 0)),
        out_shape=jax.ShapeDtypeStruct((N, H), _f32),
    )(ucat, w1, b1, w2, b2)


def _tc_pre(h, ucat, wa, wb, wcd, wcs, m1b, ua, ucu, u1b):
    blk = 1000

    def body(h_ref, u_ref, wa_ref, wb_ref, wcd_ref, wcs_ref, m1b_ref,
             ua_ref, ucu_ref, u1b_ref, a_ref, b_ref, p_ref):
        hv = h_ref[...]
        uv = u_ref[...]
        a_ref[...] = (jnp.dot(hv, wa_ref[...], preferred_element_type=_f32)
                      + jnp.dot(uv, wcd_ref[...], preferred_element_type=_f32)
                      + m1b_ref[...])
        b_ref[...] = (jnp.dot(hv, wb_ref[...], preferred_element_type=_f32)
                      + jnp.dot(uv, wcs_ref[...], preferred_element_type=_f32))
        p_ref[...] = (jnp.dot(hv, ua_ref[...], preferred_element_type=_f32)
                      + jnp.dot(uv, ucu_ref[...], preferred_element_type=_f32)
                      + u1b_ref[...])

    wspec = lambda shape: pl.BlockSpec(shape, lambda i: (0, 0))
    return pl.pallas_call(
        body,
        grid=(N // blk,),
        in_specs=[
            pl.BlockSpec((blk, H), lambda i: (i, 0)),
            pl.BlockSpec((blk, 32), lambda i: (i, 0)),
            wspec((H, H)), wspec((H, H)), wspec((32, H)), wspec((32, H)),
            wspec((1, H)), wspec((H, H)), wspec((32, H)), wspec((1, H)),
        ],
        out_specs=[pl.BlockSpec((blk, H), lambda i: (i, 0))] * 3,
        out_shape=[jax.ShapeDtypeStruct((N, H), _f32)] * 3,
    )(h, ucat, wa, wb, wcd, wcs, m1b, ua, ucu, u1b)


def _tc_edge(ad, bs, m2w, m2b):
    blk = 2048

    def body(a_ref, b_ref, w_ref, bias_ref, m_ref):
        t = _swish(a_ref[...] + b_ref[...])
        y = jnp.dot(t, w_ref[...], preferred_element_type=_f32) + bias_ref[...]
        m_ref[...] = _swish(y)

    return pl.pallas_call(
        body,
        grid=(EPAD // blk,),
        in_specs=[
            pl.BlockSpec((blk, H), lambda i: (i, 0)),
            pl.BlockSpec((blk, H), lambda i: (i, 0)),
            pl.BlockSpec((H, H), lambda i: (0, 0)),
            pl.BlockSpec((1, H), lambda i: (0, 0)),
        ],
        out_specs=pl.BlockSpec((blk, H), lambda i: (i, 0)),
        out_shape=jax.ShapeDtypeStruct((EPAD, H), _f32),
    )(ad, bs, m2w, m2b)


def _tc_upd(h, preu, p_parts, degp, ub, u2w, u2b):
    def body(h_ref, pre_ref, p_ref, dg_ref, ub_ref, u2w_ref, u2b_ref, o_ref):
        deg = jnp.maximum(dg_ref[0, :N] + dg_ref[1, :N], 1.0)
        agg = (p_ref[0, :N] + p_ref[1, :N]) * (1.0 / deg)
        x = pre_ref[...] + jnp.dot(agg, ub_ref[...], preferred_element_type=_f32)
        x = _swish(x)
        y = jnp.dot(x, u2w_ref[...], preferred_element_type=_f32) + u2b_ref[...]
        hn = h_ref[...] + _swish(y)
        mean = jnp.mean(hn, axis=0, keepdims=True)
        d = hn - mean
        var = jnp.mean(d * d, axis=0, keepdims=True)
        o_ref[...] = d / jnp.sqrt(var + EPS)

    return pl.pallas_call(
        body,
        out_shape=jax.ShapeDtypeStruct((N, H), _f32),
    )(h, preu, p_parts, degp, ub, u2w, u2b)


def _tc_decoder(h, m1f, c1b, m2f, b2row, u):
    blk = 1000

    def body(h_ref, m1_ref, c1_ref, m2_ref, b2_ref, u_ref, o_ref):
        z = jnp.dot(h_ref[...], m1_ref[...], preferred_element_type=_f32)
        z = _swish(z + c1_ref[...])
        diff = jnp.dot(z, m2_ref[...], preferred_element_type=_f32) + b2_ref[...]
        dtv = DT * (lax.broadcasted_iota(jnp.int32, (blk, TW), 1)
                    .astype(_f32) + 1.0)
        o_ref[...] = u_ref[:, TW - 1:TW] + dtv * diff

    return pl.pallas_call(
        body,
        grid=(N // blk,),
        in_specs=[
            pl.BlockSpec((blk, H), lambda i: (i, 0)),
            pl.BlockSpec((H, 304), lambda i: (0, 0)),
            pl.BlockSpec((1, 304), lambda i: (0, 0)),
            pl.BlockSpec((304, TW), lambda i: (0, 0)),
            pl.BlockSpec((1, TW), lambda i: (0, 0)),
            pl.BlockSpec((blk, TW), lambda i: (i, 0)),
        ],
        out_specs=pl.BlockSpec((blk, TW), lambda i: (i, 0)),
        out_shape=jax.ShapeDtypeStruct((N, TW), _f32),
    )(h, m1f, c1b, m2f, b2row, u)


def _sc_gather(a, b, dstg3, srcg3):
    mesh = plsc.VectorSubcoreMesh(core_axis_name="c", subcore_axis_name="s")

    @functools.partial(
        pl.kernel,
        mesh=mesh,
        out_type=[jax.ShapeDtypeStruct((EPAD, H), _f32)] * 2,
        scratch_types=[
            pltpu.VMEM((CPW, CHUNK), jnp.int32),
            pltpu.VMEM((CPW, CHUNK), jnp.int32),
            pltpu.VMEM((2, CHUNK, H), _f32),
            pltpu.VMEM((2, CHUNK, H), _f32),
            pltpu.SemaphoreType.DMA,
            pltpu.SemaphoreType.DMA,
        ],
    )
    def k(a_h, b_h, d3_h, s3_h, ad_h, bs_h, idx_d, idx_s, buf_a, buf_b, s0, s1):
        c = lax.axis_index("c")
        s = lax.axis_index("s")
        w = s * NC + c
        pltpu.sync_copy(d3_h.at[w], idx_d)
        pltpu.sync_copy(s3_h.at[w], idx_s)
        base = w * EPW
        sems = (s0, s1)

        def issue(g, slot):
            pltpu.async_copy(a_h.at[idx_d.at[g]], buf_a.at[slot], sems[slot])
            pltpu.async_copy(b_h.at[idx_s.at[g]], buf_b.at[slot], sems[slot])

        def drain(slot, cur):
            pltpu.make_async_copy(a_h.at[idx_d.at[0]], buf_a.at[slot],
                                  sems[slot]).wait()
            pltpu.make_async_copy(b_h.at[idx_s.at[0]], buf_b.at[slot],
                                  sems[slot]).wait()
            pltpu.sync_copy(buf_a.at[slot],
                            ad_h.at[pl.ds(base + cur * CHUNK, CHUNK)])
            pltpu.sync_copy(buf_b.at[slot],
                            bs_h.at[pl.ds(base + cur * CHUNK, CHUNK)])

        issue(0, 0)
        issue(1, 1)

        @pl.loop(0, CPW, step=2)
        def _(g):
            for bslot in range(2):
                cur = g + bslot
                drain(bslot, cur)

                @pl.when(cur + 2 < CPW)
                def _():
                    issue(cur + 2, bslot)

    return k(a, b, dstg3, srcg3)


def _sc_scatter(m, dst3, zeros_nd):
    mesh = plsc.VectorSubcoreMesh(core_axis_name="c", subcore_axis_name="s")

    @functools.partial(
        pl.kernel,
        mesh=mesh,
        out_type=jax.ShapeDtypeStruct((2, NPAD, H), _f32),
        scratch_types=[
            pltpu.VMEM((CPW, CHUNK), jnp.int32),
            pltpu.VMEM((2, CHUNK, H), _f32),
            pltpu.VMEM_SHARED((NPAD, H), _f32),
            pltpu.SemaphoreType.DMA,
            pltpu.SemaphoreType.DMA,
        ],
    )
    def k(m_h, d3_h, z_h, out_h, idx_d, mbuf, acc, s0, s1):
        c = lax.axis_index("c")
        s = lax.axis_index("s")
        w = s * NC + c
        pltpu.sync_copy(d3_h.at[w], idx_d)
        pltpu.sync_copy(z_h.at[pl.ds(s * NPT, NPT)], acc.at[pl.ds(s * NPT, NPT)])
        plsc.subcore_barrier()
        base = w * EPW
        sems = (s0, s1)

        def load(g, slot):
            pltpu.async_copy(m_h.at[pl.ds(base + g * CHUNK, CHUNK)],
                             mbuf.at[slot], sems[slot])

        def drain(slot, cur):
            pltpu.make_async_copy(m_h.at[pl.ds(0, CHUNK)],
                                  mbuf.at[slot], sems[slot]).wait()
            pltpu.sync_copy(mbuf.at[slot], acc.at[idx_d.at[cur]], add=True)

        load(0, 0)
        load(1, 1)

        @pl.loop(0, CPW, step=2)
        def _(g):
            for bslot in range(2):
                cur = g + bslot
                drain(bslot, cur)

                @pl.when(cur + 2 < CPW)
                def _():
                    load(cur + 2, bslot)

        plsc.subcore_barrier()
        pltpu.sync_copy(acc.at[pl.ds(s * NPT, NPT)],
                        out_h.at[c, pl.ds(s * NPT, NPT)])

    return k(m, dst3, zeros_nd)


def _sc_degree(dst3, zeros_nw, ones_cw):
    mesh = plsc.VectorSubcoreMesh(core_axis_name="c", subcore_axis_name="s")
    DW = 128

    @functools.partial(
        pl.kernel,
        mesh=mesh,
        out_type=jax.ShapeDtypeStruct((2, NPAD, DW), _f32),
        scratch_types=[
            pltpu.VMEM((CPW, CHUNK), jnp.int32),
            pltpu.VMEM((CHUNK, DW), _f32),
            pltpu.VMEM_SHARED((NPAD, DW), _f32),
        ],
    )
    def k(d3_h, z_h, o_h, out_h, idx_d, ones_v, acc):
        c = lax.axis_index("c")
        s = lax.axis_index("s")
        w = s * NC + c
        pltpu.sync_copy(d3_h.at[w], idx_d)
        pltpu.sync_copy(o_h, ones_v)
        pltpu.sync_copy(z_h.at[pl.ds(s * NPT, NPT)], acc.at[pl.ds(s * NPT, NPT)])
        plsc.subcore_barrier()

        @pl.loop(0, CPW)
        def _(g):
            pltpu.sync_copy(ones_v, acc.at[idx_d.at[g]], add=True)

        plsc.subcore_barrier()
        pltpu.sync_copy(acc.at[pl.ds(s * NPT, NPT)],
                        out_h.at[c, pl.ds(s * NPT, NPT)])

    return k(dst3, zeros_nw, ones_cw)


def kernel(u, pos, edge_index, batch, params):
    del batch
    pos_x = pos[:, 1:2] / L_PDE
    variables = pos[:, 0:1] / TMAX
    ucat = jnp.concatenate(
        [u, pos_x, variables, jnp.zeros((N, 5), _f32)], axis=1)

    w1p = jnp.concatenate(
        [params['emb_W1'], jnp.zeros((5, H), _f32)], axis=0)
    h = _tc_embed(ucat, w1p, params['emb_b1'][None, :],
                  params['emb_W2'], params['emb_b2'][None, :])

    src = edge_index[0]
    dst = edge_index[1]
    npad_e = EPAD - E
    dstg3 = jnp.concatenate(
        [dst, jnp.zeros((npad_e,), jnp.int32)]).reshape(NW, CPW, CHUNK)
    srcg3 = jnp.concatenate(
        [src, jnp.zeros((npad_e,), jnp.int32)]).reshape(NW, CPW, CHUNK)
    dsts3 = jnp.concatenate(
        [dst, jnp.full((npad_e,), DUMMY, jnp.int32)]).reshape(NW, CPW, CHUNK)

    zeros_nd = jnp.zeros((NPAD, H), _f32)
    zeros_nw = jnp.zeros((NPAD, 128), _f32)
    ones_cw = jnp.ones((CHUNK, 128), _f32)
    degp = _sc_degree(dsts3, zeros_nw, ones_cw)[:, :, 0:1]

    pad5 = jnp.zeros((5, H), _f32)
    pad1 = jnp.zeros((1, H), _f32)
    for lp in params['layers']:
        w = lp['m1W']
        wa = w[0:H]
        wb = w[H:2 * H]
        wc = w[2 * H:2 * H + TW]
        wd = w[2 * H + TW:2 * H + TW + 1]
        we = w[2 * H + TW + 1:2 * H + TW + 2]
        wcd = jnp.concatenate([wc, wd, we, pad5], axis=0)
        wcs = jnp.concatenate([-wc, -wd, pad1, pad5], axis=0)
        uw = lp['u1W']
        ua = uw[0:H]
        ub = uw[H:2 * H]
        uc = uw[2 * H:2 * H + 1]
        ucu = jnp.concatenate([jnp.zeros((TW + 1, H), _f32), uc, pad5], axis=0)

        a, b, preu = _tc_pre(h, ucat, wa, wb, wcd, wcs, lp['m1b'][None, :],
                             ua, ucu, lp['u1b'][None, :])
        ad, bs = _sc_gather(a, b, dstg3, srcg3)
        m = _tc_edge(ad, bs, lp['m2W'], lp['m2b'][None, :])
        p_parts = _sc_scatter(m, dsts3, zeros_nd)
        h = _tc_upd(h, preu, p_parts, degp, ub, lp['u2W'],
                    lp['u2b'][None, :])

    P1 = (H - 16) // 3 + 1
    s1 = np.zeros((P1, H, 16), np.float32)
    for p in range(P1):
        for kk in range(16):
            s1[p, 3 * p + kk, kk] = 1.0
    s2 = np.zeros((TW, P1, 14), np.float32)
    for q in range(TW):
        for kk in range(14):
            s2[q, q + kk, kk] = 1.0
    m1f = jnp.einsum('pjk,ck->jpc', jnp.asarray(s1),
                     params['conv_W1'][:, 0, :]).reshape(H, P1 * 8)
    m2f = jnp.einsum('qpk,ck->pcq', jnp.asarray(s2),
                     params['conv_W2'][0]).reshape(P1 * 8, TW)
    c1b = jnp.tile(params['conv_b1'], P1)[None, :]
    b2row = jnp.broadcast_to(params['conv_b2'], (TW,))[None, :]
    return _tc_decoder(h, m1f, c1b, m2f, b2row, u)


# speedup vs baseline: 1.6871x; 1.6513x over previous
import functools

import jax
import jax.numpy as jnp
import numpy as np
from jax import lax
from jax.experimental import pallas as pl
from jax.experimental.pallas import tpu as pltpu
from jax.experimental.pallas import tpu_sc as plsc

N = 10000
E = 320000
TW = 25
H = 128
L_PDE = 16.0
TMAX = 4.0
DT = 0.01
EPS = 1e-5

NC = 2
NS = 16
NW = NC * NS
CHUNK = 80
EPAD = E
EPW = EPAD // NW
CPW = EPW // CHUNK
NPAD = 10240
NPT = NPAD // NS


_f32 = jnp.float32


def _swish(x):
    return x * (1.0 / (1.0 + jnp.exp(-x)))


def _tc_embed(ucat, w1, b1, w2, b2):
    blk = 1000

    def body(u_ref, w1_ref, b1_ref, w2_ref, b2_ref, h_ref):
        x = jnp.dot(u_ref[...], w1_ref[...], preferred_element_type=_f32)
        x = _swish(x + b1_ref[...])
        y = jnp.dot(x, w2_ref[...], preferred_element_type=_f32)
        h_ref[...] = _swish(y + b2_ref[...])

    return pl.pallas_call(
        body,
        grid=(N // blk,),
        in_specs=[
            pl.BlockSpec((blk, 32), lambda i: (i, 0)),
            pl.BlockSpec((32, H), lambda i: (0, 0)),
            pl.BlockSpec((1, H), lambda i: (0, 0)),
            pl.BlockSpec((H, H), lambda i: (0, 0)),
            pl.BlockSpec((1, H), lambda i: (0, 0)),
        ],
        out_specs=pl.BlockSpec((blk, H), lambda i: (i, 0)),
        out_shape=jax.ShapeDtypeStruct((N, H), _f32),
    )(ucat, w1, b1, w2, b2)


def _tc_pre(h, ucat, wa, wb, wcd, wcs, m1b, ua, ucu, u1b):
    blk = 1000

    def body(h_ref, u_ref, wa_ref, wb_ref, wcd_ref, wcs_ref, m1b_ref,
             ua_ref, ucu_ref, u1b_ref, a_ref, b_ref, p_ref):
        hv = h_ref[...]
        uv = u_ref[...]
        a_ref[...] = (jnp.dot(hv, wa_ref[...], preferred_element_type=_f32)
                      + jnp.dot(uv, wcd_ref[...], preferred_element_type=_f32)
                      + m1b_ref[...])
        b_ref[...] = (jnp.dot(hv, wb_ref[...], preferred_element_type=_f32)
                      + jnp.dot(uv, wcs_ref[...], preferred_element_type=_f32))
        p_ref[...] = (jnp.dot(hv, ua_ref[...], preferred_element_type=_f32)
                      + jnp.dot(uv, ucu_ref[...], preferred_element_type=_f32)
                      + u1b_ref[...])

    wspec = lambda shape: pl.BlockSpec(shape, lambda i: (0, 0))
    return pl.pallas_call(
        body,
        grid=(N // blk,),
        in_specs=[
            pl.BlockSpec((blk, H), lambda i: (i, 0)),
            pl.BlockSpec((blk, 32), lambda i: (i, 0)),
            wspec((H, H)), wspec((H, H)), wspec((32, H)), wspec((32, H)),
            wspec((1, H)), wspec((H, H)), wspec((32, H)), wspec((1, H)),
        ],
        out_specs=[pl.BlockSpec((blk, H), lambda i: (i, 0))] * 3,
        out_shape=[jax.ShapeDtypeStruct((N, H), _f32)] * 3,
    )(h, ucat, wa, wb, wcd, wcs, m1b, ua, ucu, u1b)


def _tc_edge(ad, bs, m2w, m2b):
    blk = 2000

    def body(a_ref, b_ref, w_ref, bias_ref, m_ref):
        t = _swish(a_ref[...] + b_ref[...])
        y = jnp.dot(t, w_ref[...], preferred_element_type=_f32) + bias_ref[...]
        m_ref[...] = _swish(y)

    return pl.pallas_call(
        body,
        grid=(EPAD // blk,),
        in_specs=[
            pl.BlockSpec((blk, H), lambda i: (i, 0)),
            pl.BlockSpec((blk, H), lambda i: (i, 0)),
            pl.BlockSpec((H, H), lambda i: (0, 0)),
            pl.BlockSpec((1, H), lambda i: (0, 0)),
        ],
        out_specs=pl.BlockSpec((blk, H), lambda i: (i, 0)),
        out_shape=jax.ShapeDtypeStruct((EPAD, H), _f32),
    )(ad, bs, m2w, m2b)


def _tc_upd(h, preu, p_parts, degp, ub, u2w, u2b):
    def body(h_ref, pre_ref, p_ref, dg_ref, ub_ref, u2w_ref, u2b_ref, o_ref):
        deg = jnp.maximum(dg_ref[0, :N] + dg_ref[1, :N], 1.0)
        agg = (p_ref[0, :N] + p_ref[1, :N]) * (1.0 / deg)
        x = pre_ref[...] + jnp.dot(agg, ub_ref[...], preferred_element_type=_f32)
        x = _swish(x)
        y = jnp.dot(x, u2w_ref[...], preferred_element_type=_f32) + u2b_ref[...]
        hn = h_ref[...] + _swish(y)
        mean = jnp.mean(hn, axis=0, keepdims=True)
        d = hn - mean
        var = jnp.mean(d * d, axis=0, keepdims=True)
        o_ref[...] = d / jnp.sqrt(var + EPS)

    return pl.pallas_call(
        body,
        out_shape=jax.ShapeDtypeStruct((N, H), _f32),
    )(h, preu, p_parts, degp, ub, u2w, u2b)


def _tc_decoder(h, m1f, c1b, m2f, b2row, u):
    blk = 1000

    def body(h_ref, m1_ref, c1_ref, m2_ref, b2_ref, u_ref, o_ref):
        z = jnp.dot(h_ref[...], m1_ref[...], preferred_element_type=_f32)
        z = _swish(z + c1_ref[...])
        diff = jnp.dot(z, m2_ref[...], preferred_element_type=_f32) + b2_ref[...]
        dtv = DT * (lax.broadcasted_iota(jnp.int32, (blk, TW), 1)
                    .astype(_f32) + 1.0)
        o_ref[...] = u_ref[:, TW - 1:TW] + dtv * diff

    return pl.pallas_call(
        body,
        grid=(N // blk,),
        in_specs=[
            pl.BlockSpec((blk, H), lambda i: (i, 0)),
            pl.BlockSpec((H, 304), lambda i: (0, 0)),
            pl.BlockSpec((1, 304), lambda i: (0, 0)),
            pl.BlockSpec((304, TW), lambda i: (0, 0)),
            pl.BlockSpec((1, TW), lambda i: (0, 0)),
            pl.BlockSpec((blk, TW), lambda i: (i, 0)),
        ],
        out_specs=pl.BlockSpec((blk, TW), lambda i: (i, 0)),
        out_shape=jax.ShapeDtypeStruct((N, TW), _f32),
    )(h, m1f, c1b, m2f, b2row, u)


def _sc_gather(a, b, dstg3, srcg3):
    mesh = plsc.VectorSubcoreMesh(core_axis_name="c", subcore_axis_name="s")

    @functools.partial(
        pl.kernel,
        mesh=mesh,
        out_type=[jax.ShapeDtypeStruct((EPAD, H), _f32)] * 2,
        scratch_types=[
            pltpu.VMEM((CPW, CHUNK), jnp.int32),
            pltpu.VMEM((CPW, CHUNK), jnp.int32),
            pltpu.VMEM((2, CHUNK, H), _f32),
            pltpu.VMEM((2, CHUNK, H), _f32),
            pltpu.SemaphoreType.DMA,
            pltpu.SemaphoreType.DMA,
        ],
    )
    def k(a_h, b_h, d3_h, s3_h, ad_h, bs_h, idx_d, idx_s, buf_a, buf_b, s0, s1):
        c = lax.axis_index("c")
        s = lax.axis_index("s")
        w = s * NC + c
        pltpu.sync_copy(d3_h.at[w], idx_d)
        pltpu.sync_copy(s3_h.at[w], idx_s)
        base = w * EPW
        sems = (s0, s1)

        def issue(g, slot):
            pltpu.async_copy(a_h.at[idx_d.at[g]], buf_a.at[slot], sems[slot])
            pltpu.async_copy(b_h.at[idx_s.at[g]], buf_b.at[slot], sems[slot])

        def drain(slot, cur):
            pltpu.make_async_copy(a_h.at[idx_d.at[0]], buf_a.at[slot],
                                  sems[slot]).wait()
            pltpu.make_async_copy(b_h.at[idx_s.at[0]], buf_b.at[slot],
                                  sems[slot]).wait()
            pltpu.sync_copy(buf_a.at[slot],
                            ad_h.at[pl.ds(base + cur * CHUNK, CHUNK)])
            pltpu.sync_copy(buf_b.at[slot],
                            bs_h.at[pl.ds(base + cur * CHUNK, CHUNK)])

        issue(0, 0)
        issue(1, 1)

        @pl.loop(0, CPW - 1, step=2)
        def _(g):
            for bslot in range(2):
                cur = g + bslot
                drain(bslot, cur)

                @pl.when(cur + 2 < CPW)
                def _():
                    issue(cur + 2, bslot)

        drain((CPW - 1) % 2, CPW - 1)

    return k(a, b, dstg3, srcg3)


def _sc_scatter(m, dst3, zeros_nd):
    mesh = plsc.VectorSubcoreMesh(core_axis_name="c", subcore_axis_name="s")

    @functools.partial(
        pl.kernel,
        mesh=mesh,
        out_type=jax.ShapeDtypeStruct((2, NPAD, H), _f32),
        scratch_types=[
            pltpu.VMEM((CPW, CHUNK), jnp.int32),
            pltpu.VMEM((2, CHUNK, H), _f32),
            pltpu.VMEM_SHARED((NPAD, H), _f32),
            pltpu.SemaphoreType.DMA,
            pltpu.SemaphoreType.DMA,
        ],
    )
    def k(m_h, d3_h, z_h, out_h, idx_d, mbuf, acc, s0, s1):
        c = lax.axis_index("c")
        s = lax.axis_index("s")
        w = s * NC + c
        pltpu.sync_copy(d3_h.at[w], idx_d)
        pltpu.sync_copy(z_h.at[pl.ds(s * NPT, NPT)], acc.at[pl.ds(s * NPT, NPT)])
        plsc.subcore_barrier()
        base = w * EPW
        sems = (s0, s1)

        def load(g, slot):
            pltpu.async_copy(m_h.at[pl.ds(base + g * CHUNK, CHUNK)],
                             mbuf.at[slot], sems[slot])

        def drain(slot, cur):
            pltpu.make_async_copy(m_h.at[pl.ds(0, CHUNK)],
                                  mbuf.at[slot], sems[slot]).wait()
            pltpu.sync_copy(mbuf.at[slot], acc.at[idx_d.at[cur]], add=True)

        load(0, 0)
        load(1, 1)

        @pl.loop(0, CPW - 1, step=2)
        def _(g):
            for bslot in range(2):
                cur = g + bslot
                drain(bslot, cur)

                @pl.when(cur + 2 < CPW)
                def _():
                    load(cur + 2, bslot)

        drain((CPW - 1) % 2, CPW - 1)

        plsc.subcore_barrier()
        pltpu.sync_copy(acc.at[pl.ds(s * NPT, NPT)],
                        out_h.at[c, pl.ds(s * NPT, NPT)])

    return k(m, dst3, zeros_nd)


def _sc_degree(dst3, zeros_nw, ones_cw):
    mesh = plsc.VectorSubcoreMesh(core_axis_name="c", subcore_axis_name="s")
    DW = 128

    @functools.partial(
        pl.kernel,
        mesh=mesh,
        out_type=jax.ShapeDtypeStruct((2, NPAD, DW), _f32),
        scratch_types=[
            pltpu.VMEM((CPW, CHUNK), jnp.int32),
            pltpu.VMEM((CHUNK, DW), _f32),
            pltpu.VMEM_SHARED((NPAD, DW), _f32),
        ],
    )
    def k(d3_h, z_h, o_h, out_h, idx_d, ones_v, acc):
        c = lax.axis_index("c")
        s = lax.axis_index("s")
        w = s * NC + c
        pltpu.sync_copy(d3_h.at[w], idx_d)
        pltpu.sync_copy(o_h, ones_v)
        pltpu.sync_copy(z_h.at[pl.ds(s * NPT, NPT)], acc.at[pl.ds(s * NPT, NPT)])
        plsc.subcore_barrier()

        @pl.loop(0, CPW)
        def _(g):
            pltpu.sync_copy(ones_v, acc.at[idx_d.at[g]], add=True)

        plsc.subcore_barrier()
        pltpu.sync_copy(acc.at[pl.ds(s * NPT, NPT)],
                        out_h.at[c, pl.ds(s * NPT, NPT)])

    return k(dst3, zeros_nw, ones_cw)


def kernel(u, pos, edge_index, batch, params):
    del batch
    pos_x = pos[:, 1:2] / L_PDE
    variables = pos[:, 0:1] / TMAX
    ucat = jnp.concatenate(
        [u, pos_x, variables, jnp.zeros((N, 5), _f32)], axis=1)

    w1p = jnp.concatenate(
        [params['emb_W1'], jnp.zeros((5, H), _f32)], axis=0)
    h = _tc_embed(ucat, w1p, params['emb_b1'][None, :],
                  params['emb_W2'], params['emb_b2'][None, :])

    src = edge_index[0]
    dst = edge_index[1]
    dstg3 = dst.reshape(NW, CPW, CHUNK)
    srcg3 = src.reshape(NW, CPW, CHUNK)
    dsts3 = dstg3

    zeros_nd = jnp.zeros((NPAD, H), _f32)
    zeros_nw = jnp.zeros((NPAD, 128), _f32)
    ones_cw = jnp.ones((CHUNK, 128), _f32)
    degp = _sc_degree(dsts3, zeros_nw, ones_cw)[:, :, 0:1]

    pad5 = jnp.zeros((5, H), _f32)
    pad1 = jnp.zeros((1, H), _f32)
    for lp in params['layers']:
        w = lp['m1W']
        wa = w[0:H]
        wb = w[H:2 * H]
        wc = w[2 * H:2 * H + TW]
        wd = w[2 * H + TW:2 * H + TW + 1]
        we = w[2 * H + TW + 1:2 * H + TW + 2]
        wcd = jnp.concatenate([wc, wd, we, pad5], axis=0)
        wcs = jnp.concatenate([-wc, -wd, pad1, pad5], axis=0)
        uw = lp['u1W']
        ua = uw[0:H]
        ub = uw[H:2 * H]
        uc = uw[2 * H:2 * H + 1]
        ucu = jnp.concatenate([jnp.zeros((TW + 1, H), _f32), uc, pad5], axis=0)

        a, b, preu = _tc_pre(h, ucat, wa, wb, wcd, wcs, lp['m1b'][None, :],
                             ua, ucu, lp['u1b'][None, :])
        ad, bs = _sc_gather(a, b, dstg3, srcg3)
        m = _tc_edge(ad, bs, lp['m2W'], lp['m2b'][None, :])
        p_parts = _sc_scatter(m, dsts3, zeros_nd)
        h = _tc_upd(h, preu, p_parts, degp, ub, lp['u2W'],
                    lp['u2b'][None, :])

    P1 = (H - 16) // 3 + 1
    s1 = np.zeros((P1, H, 16), np.float32)
    for p in range(P1):
        for kk in range(16):
            s1[p, 3 * p + kk, kk] = 1.0
    s2 = np.zeros((TW, P1, 14), np.float32)
    for q in range(TW):
        for kk in range(14):
            s2[q, q + kk, kk] = 1.0
    m1f = jnp.einsum('pjk,ck->jpc', jnp.asarray(s1),
                     params['conv_W1'][:, 0, :]).reshape(H, P1 * 8)
    m2f = jnp.einsum('qpk,ck->pcq', jnp.asarray(s2),
                     params['conv_W2'][0]).reshape(P1 * 8, TW)
    c1b = jnp.tile(params['conv_b1'], P1)[None, :]
    b2row = jnp.broadcast_to(params['conv_b2'], (TW,))[None, :]
    return _tc_decoder(h, m1f, c1b, m2f, b2row, u)


# speedup vs baseline: 1.9584x; 1.1608x over previous
import functools

import jax
import jax.numpy as jnp
import numpy as np
from jax import lax
from jax.experimental import pallas as pl
from jax.experimental.pallas import tpu as pltpu
from jax.experimental.pallas import tpu_sc as plsc

N = 10000
E = 320000
TW = 25
H = 128
L_PDE = 16.0
TMAX = 4.0
DT = 0.01
EPS = 1e-5

NC = 2
NS = 16
NW = NC * NS
CHUNK = 80
EPAD = E
EPW = EPAD // NW
CPW = EPW // CHUNK
NPAD = 10240
NPT = NPAD // NS


_f32 = jnp.float32


def _swish(x):
    return x * (1.0 / (1.0 + jnp.exp(-x)))


def _tc_embed(ucat, w1, b1, w2, b2):
    blk = 1000

    def body(u_ref, w1_ref, b1_ref, w2_ref, b2_ref, h_ref):
        x = jnp.dot(u_ref[...], w1_ref[...], preferred_element_type=_f32)
        x = _swish(x + b1_ref[...])
        y = jnp.dot(x, w2_ref[...], preferred_element_type=_f32)
        h_ref[...] = _swish(y + b2_ref[...])

    return pl.pallas_call(
        body,
        grid=(N // blk,),
        in_specs=[
            pl.BlockSpec((blk, 32), lambda i: (i, 0)),
            pl.BlockSpec((32, H), lambda i: (0, 0)),
            pl.BlockSpec((1, H), lambda i: (0, 0)),
            pl.BlockSpec((H, H), lambda i: (0, 0)),
            pl.BlockSpec((1, H), lambda i: (0, 0)),
        ],
        out_specs=pl.BlockSpec((blk, H), lambda i: (i, 0)),
        out_shape=jax.ShapeDtypeStruct((N, H), _f32),
    )(ucat, w1, b1, w2, b2)


def _tc_pre(h, ucat, wa, wb, wcd, wcs, m1b, ua, ucu, u1b):
    blk = 1000

    def body(h_ref, u_ref, wa_ref, wb_ref, wcd_ref, wcs_ref, m1b_ref,
             ua_ref, ucu_ref, u1b_ref, a_ref, b_ref, p_ref):
        hv = h_ref[...]
        uv = u_ref[...]
        a_ref[...] = (jnp.dot(hv, wa_ref[...], preferred_element_type=_f32)
                      + jnp.dot(uv, wcd_ref[...], preferred_element_type=_f32)
                      + m1b_ref[...])
        b_ref[...] = (jnp.dot(hv, wb_ref[...], preferred_element_type=_f32)
                      + jnp.dot(uv, wcs_ref[...], preferred_element_type=_f32))
        p_ref[...] = (jnp.dot(hv, ua_ref[...], preferred_element_type=_f32)
                      + jnp.dot(uv, ucu_ref[...], preferred_element_type=_f32)
                      + u1b_ref[...])

    wspec = lambda shape: pl.BlockSpec(shape, lambda i: (0, 0))
    return pl.pallas_call(
        body,
        grid=(N // blk,),
        in_specs=[
            pl.BlockSpec((blk, H), lambda i: (i, 0)),
            pl.BlockSpec((blk, 32), lambda i: (i, 0)),
            wspec((H, H)), wspec((H, H)), wspec((32, H)), wspec((32, H)),
            wspec((1, H)), wspec((H, H)), wspec((32, H)), wspec((1, H)),
        ],
        out_specs=[pl.BlockSpec((blk, H), lambda i: (i, 0))] * 3,
        out_shape=[jax.ShapeDtypeStruct((N, H), _f32)] * 3,
    )(h, ucat, wa, wb, wcd, wcs, m1b, ua, ucu, u1b)


def _tc_edge(ad, bs, m2w, m2b):
    blk = 2000

    def body(a_ref, b_ref, w_ref, bias_ref, m_ref):
        t = _swish(a_ref[...] + b_ref[...])
        y = jnp.dot(t, w_ref[...], preferred_element_type=_f32) + bias_ref[...]
        m_ref[...] = _swish(y)

    return pl.pallas_call(
        body,
        grid=(EPAD // blk,),
        in_specs=[
            pl.BlockSpec((blk, H), lambda i: (i, 0)),
            pl.BlockSpec((blk, H), lambda i: (i, 0)),
            pl.BlockSpec((H, H), lambda i: (0, 0)),
            pl.BlockSpec((1, H), lambda i: (0, 0)),
        ],
        out_specs=pl.BlockSpec((blk, H), lambda i: (i, 0)),
        out_shape=jax.ShapeDtypeStruct((EPAD, H), _f32),
    )(ad, bs, m2w, m2b)


def _tc_upd(h, preu, p_parts, degp, ub, u2w, u2b):
    def body(h_ref, pre_ref, p_ref, dg_ref, ub_ref, u2w_ref, u2b_ref, o_ref):
        deg = jnp.maximum(dg_ref[0, :N] + dg_ref[1, :N], 1.0)
        agg = (p_ref[0, :N] + p_ref[1, :N]) * (1.0 / deg)
        x = pre_ref[...] + jnp.dot(agg, ub_ref[...], preferred_element_type=_f32)
        x = _swish(x)
        y = jnp.dot(x, u2w_ref[...], preferred_element_type=_f32) + u2b_ref[...]
        hn = h_ref[...] + _swish(y)
        mean = jnp.mean(hn, axis=0, keepdims=True)
        d = hn - mean
        var = jnp.mean(d * d, axis=0, keepdims=True)
        o_ref[...] = d / jnp.sqrt(var + EPS)

    return pl.pallas_call(
        body,
        out_shape=jax.ShapeDtypeStruct((N, H), _f32),
    )(h, preu, p_parts, degp, ub, u2w, u2b)


def _tc_decoder(h, m1f, c1b, m2f, b2row, u):
    blk = 1000

    def body(h_ref, m1_ref, c1_ref, m2_ref, b2_ref, u_ref, o_ref):
        z = jnp.dot(h_ref[...], m1_ref[...], preferred_element_type=_f32)
        z = _swish(z + c1_ref[...])
        diff = jnp.dot(z, m2_ref[...], preferred_element_type=_f32) + b2_ref[...]
        dtv = DT * (lax.broadcasted_iota(jnp.int32, (blk, TW), 1)
                    .astype(_f32) + 1.0)
        o_ref[...] = u_ref[:, TW - 1:TW] + dtv * diff

    return pl.pallas_call(
        body,
        grid=(N // blk,),
        in_specs=[
            pl.BlockSpec((blk, H), lambda i: (i, 0)),
            pl.BlockSpec((H, 304), lambda i: (0, 0)),
            pl.BlockSpec((1, 304), lambda i: (0, 0)),
            pl.BlockSpec((304, TW), lambda i: (0, 0)),
            pl.BlockSpec((1, TW), lambda i: (0, 0)),
            pl.BlockSpec((blk, TW), lambda i: (i, 0)),
        ],
        out_specs=pl.BlockSpec((blk, TW), lambda i: (i, 0)),
        out_shape=jax.ShapeDtypeStruct((N, TW), _f32),
    )(h, m1f, c1b, m2f, b2row, u)


def _sc_gather_add(a, b, dstg3, srcg3):
    mesh = plsc.VectorSubcoreMesh(core_axis_name="c", subcore_axis_name="s")

    @functools.partial(
        pl.kernel,
        mesh=mesh,
        out_type=jax.ShapeDtypeStruct((EPAD, H), _f32),
        scratch_types=[
            pltpu.VMEM((CPW, CHUNK), jnp.int32),
            pltpu.VMEM((CPW, CHUNK), jnp.int32),
            pltpu.VMEM((4, CHUNK, H), _f32),
            pltpu.SemaphoreType.DMA,
            pltpu.SemaphoreType.DMA,
            pltpu.SemaphoreType.DMA,
            pltpu.SemaphoreType.DMA,
        ],
    )
    def k(a_h, b_h, d3_h, s3_h, e_h, idx_d, idx_s, buf, s0, s1, s2, s3):
        c = lax.axis_index("c")
        s = lax.axis_index("s")
        w = s * NC + c
        pltpu.sync_copy(d3_h.at[w], idx_d)
        pltpu.sync_copy(s3_h.at[w], idx_s)
        base = w * EPW
        sems = (s0, s1, s2, s3)

        def issue_a(g, slot):
            pltpu.async_copy(a_h.at[idx_d.at[g]], buf.at[slot], sems[slot])

        def issue_b(g, slot):
            pltpu.async_copy(b_h.at[idx_s.at[g]], buf.at[slot], sems[slot],
                             add=True)

        def wait_buf(slot):
            pltpu.make_async_copy(a_h.at[idx_d.at[0]], buf.at[slot],
                                  sems[slot]).wait()

        def step(cur, j):
            wait_buf(j)
            pltpu.sync_copy(buf.at[j],
                            e_h.at[pl.ds(base + cur * CHUNK, CHUNK)])

            @pl.when(cur + 4 < CPW)
            def _():
                issue_a(cur + 4, j)

            nslot = (j + 2) % 4

            @pl.when(cur + 2 < CPW)
            def _():
                wait_buf(nslot)
                issue_b(cur + 2, nslot)

        for j in range(4):
            issue_a(j, j)
        for j in range(2):
            wait_buf(j)
            issue_b(j, j)

        @pl.loop(0, CPW - 1, step=4)
        def _(g):
            for j in range(4):
                step(g + j, j)

        step(CPW - 1, (CPW - 1) % 4)

    return k(a, b, dstg3, srcg3)


def _tc_edge_fused(e, m2w, m2b):
    blk = 2000

    def body(e_ref, w_ref, bias_ref, m_ref):
        t = _swish(e_ref[...])
        y = jnp.dot(t, w_ref[...], preferred_element_type=_f32) + bias_ref[...]
        m_ref[...] = _swish(y)

    return pl.pallas_call(
        body,
        grid=(EPAD // blk,),
        in_specs=[
            pl.BlockSpec((blk, H), lambda i: (i, 0)),
            pl.BlockSpec((H, H), lambda i: (0, 0)),
            pl.BlockSpec((1, H), lambda i: (0, 0)),
        ],
        out_specs=pl.BlockSpec((blk, H), lambda i: (i, 0)),
        out_shape=jax.ShapeDtypeStruct((EPAD, H), _f32),
    )(e, m2w, m2b)


def _sc_gather(a, b, dstg3, srcg3):
    mesh = plsc.VectorSubcoreMesh(core_axis_name="c", subcore_axis_name="s")

    @functools.partial(
        pl.kernel,
        mesh=mesh,
        out_type=[jax.ShapeDtypeStruct((EPAD, H), _f32)] * 2,
        scratch_types=[
            pltpu.VMEM((CPW, CHUNK), jnp.int32),
            pltpu.VMEM((CPW, CHUNK), jnp.int32),
            pltpu.VMEM((2, CHUNK, H), _f32),
            pltpu.VMEM((2, CHUNK, H), _f32),
            pltpu.SemaphoreType.DMA,
            pltpu.SemaphoreType.DMA,
        ],
    )
    def k(a_h, b_h, d3_h, s3_h, ad_h, bs_h, idx_d, idx_s, buf_a, buf_b, s0, s1):
        c = lax.axis_index("c")
        s = lax.axis_index("s")
        w = s * NC + c
        pltpu.sync_copy(d3_h.at[w], idx_d)
        pltpu.sync_copy(s3_h.at[w], idx_s)
        base = w * EPW
        sems = (s0, s1)

        def issue(g, slot):
            pltpu.async_copy(a_h.at[idx_d.at[g]], buf_a.at[slot], sems[slot])
            pltpu.async_copy(b_h.at[idx_s.at[g]], buf_b.at[slot], sems[slot])

        def drain(slot, cur):
            pltpu.make_async_copy(a_h.at[idx_d.at[0]], buf_a.at[slot],
                                  sems[slot]).wait()
            pltpu.make_async_copy(b_h.at[idx_s.at[0]], buf_b.at[slot],
                                  sems[slot]).wait()
            pltpu.sync_copy(buf_a.at[slot],
                            ad_h.at[pl.ds(base + cur * CHUNK, CHUNK)])
            pltpu.sync_copy(buf_b.at[slot],
                            bs_h.at[pl.ds(base + cur * CHUNK, CHUNK)])

        issue(0, 0)
        issue(1, 1)

        @pl.loop(0, CPW - 1, step=2)
        def _(g):
            for bslot in range(2):
                cur = g + bslot
                drain(bslot, cur)

                @pl.when(cur + 2 < CPW)
                def _():
                    issue(cur + 2, bslot)

        drain((CPW - 1) % 2, CPW - 1)

    return k(a, b, dstg3, srcg3)


def _sc_scatter(m, dst3, zeros_nd):
    mesh = plsc.VectorSubcoreMesh(core_axis_name="c", subcore_axis_name="s")

    @functools.partial(
        pl.kernel,
        mesh=mesh,
        out_type=jax.ShapeDtypeStruct((2, NPAD, H), _f32),
        scratch_types=[
            pltpu.VMEM((CPW, CHUNK), jnp.int32),
            pltpu.VMEM((2, CHUNK, H), _f32),
            pltpu.VMEM_SHARED((NPAD, H), _f32),
            pltpu.SemaphoreType.DMA,
            pltpu.SemaphoreType.DMA,
        ],
    )
    def k(m_h, d3_h, z_h, out_h, idx_d, mbuf, acc, s0, s1):
        c = lax.axis_index("c")
        s = lax.axis_index("s")
        w = s * NC + c
        pltpu.sync_copy(d3_h.at[w], idx_d)
        pltpu.sync_copy(z_h.at[pl.ds(s * NPT, NPT)], acc.at[pl.ds(s * NPT, NPT)])
        plsc.subcore_barrier()
        base = w * EPW
        sems = (s0, s1)

        def load(g, slot):
            pltpu.async_copy(m_h.at[pl.ds(base + g * CHUNK, CHUNK)],
                             mbuf.at[slot], sems[slot])

        def drain(slot, cur):
            pltpu.make_async_copy(m_h.at[pl.ds(0, CHUNK)],
                                  mbuf.at[slot], sems[slot]).wait()
            pltpu.sync_copy(mbuf.at[slot], acc.at[idx_d.at[cur]], add=True)

        load(0, 0)
        load(1, 1)

        @pl.loop(0, CPW - 1, step=2)
        def _(g):
            for bslot in range(2):
                cur = g + bslot
                drain(bslot, cur)

                @pl.when(cur + 2 < CPW)
                def _():
                    load(cur + 2, bslot)

        drain((CPW - 1) % 2, CPW - 1)

        plsc.subcore_barrier()
        pltpu.sync_copy(acc.at[pl.ds(s * NPT, NPT)],
                        out_h.at[c, pl.ds(s * NPT, NPT)])

    return k(m, dst3, zeros_nd)


def _sc_degree(dst3, zeros_nw, ones_cw):
    mesh = plsc.VectorSubcoreMesh(core_axis_name="c", subcore_axis_name="s")
    DW = 128

    @functools.partial(
        pl.kernel,
        mesh=mesh,
        out_type=jax.ShapeDtypeStruct((2, NPAD, DW), _f32),
        scratch_types=[
            pltpu.VMEM((CPW, CHUNK), jnp.int32),
            pltpu.VMEM((CHUNK, DW), _f32),
            pltpu.VMEM_SHARED((NPAD, DW), _f32),
        ],
    )
    def k(d3_h, z_h, o_h, out_h, idx_d, ones_v, acc):
        c = lax.axis_index("c")
        s = lax.axis_index("s")
        w = s * NC + c
        pltpu.sync_copy(d3_h.at[w], idx_d)
        pltpu.sync_copy(o_h, ones_v)
        pltpu.sync_copy(z_h.at[pl.ds(s * NPT, NPT)], acc.at[pl.ds(s * NPT, NPT)])
        plsc.subcore_barrier()

        @pl.loop(0, CPW)
        def _(g):
            pltpu.sync_copy(ones_v, acc.at[idx_d.at[g]], add=True)

        plsc.subcore_barrier()
        pltpu.sync_copy(acc.at[pl.ds(s * NPT, NPT)],
                        out_h.at[c, pl.ds(s * NPT, NPT)])

    return k(dst3, zeros_nw, ones_cw)


def kernel(u, pos, edge_index, batch, params):
    del batch
    pos_x = pos[:, 1:2] / L_PDE
    variables = pos[:, 0:1] / TMAX
    ucat = jnp.concatenate(
        [u, pos_x, variables, jnp.zeros((N, 5), _f32)], axis=1)

    w1p = jnp.concatenate(
        [params['emb_W1'], jnp.zeros((5, H), _f32)], axis=0)
    h = _tc_embed(ucat, w1p, params['emb_b1'][None, :],
                  params['emb_W2'], params['emb_b2'][None, :])

    src = edge_index[0]
    dst = edge_index[1]
    dstg3 = dst.reshape(NW, CPW, CHUNK)
    srcg3 = src.reshape(NW, CPW, CHUNK)
    dsts3 = dstg3

    zeros_nd = jnp.zeros((NPAD, H), _f32)
    zeros_nw = jnp.zeros((NPAD, 128), _f32)
    ones_cw = jnp.ones((CHUNK, 128), _f32)
    degp = _sc_degree(dsts3, zeros_nw, ones_cw)[:, :, 0:1]

    pad5 = jnp.zeros((5, H), _f32)
    pad1 = jnp.zeros((1, H), _f32)
    for lp in params['layers']:
        w = lp['m1W']
        wa = w[0:H]
        wb = w[H:2 * H]
        wc = w[2 * H:2 * H + TW]
        wd = w[2 * H + TW:2 * H + TW + 1]
        we = w[2 * H + TW + 1:2 * H + TW + 2]
        wcd = jnp.concatenate([wc, wd, we, pad5], axis=0)
        wcs = jnp.concatenate([-wc, -wd, pad1, pad5], axis=0)
        uw = lp['u1W']
        ua = uw[0:H]
        ub = uw[H:2 * H]
        uc = uw[2 * H:2 * H + 1]
        ucu = jnp.concatenate([jnp.zeros((TW + 1, H), _f32), uc, pad5], axis=0)

        a, b, preu = _tc_pre(h, ucat, wa, wb, wcd, wcs, lp['m1b'][None, :],
                             ua, ucu, lp['u1b'][None, :])
        e = _sc_gather_add(a, b, dstg3, srcg3)
        m = _tc_edge_fused(e, lp['m2W'], lp['m2b'][None, :])
        p_parts = _sc_scatter(m, dsts3, zeros_nd)
        h = _tc_upd(h, preu, p_parts, degp, ub, lp['u2W'],
                    lp['u2b'][None, :])

    P1 = (H - 16) // 3 + 1
    s1 = np.zeros((P1, H, 16), np.float32)
    for p in range(P1):
        for kk in range(16):
            s1[p, 3 * p + kk, kk] = 1.0
    s2 = np.zeros((TW, P1, 14), np.float32)
    for q in range(TW):
        for kk in range(14):
            s2[q, q + kk, kk] = 1.0
    m1f = jnp.einsum('pjk,ck->jpc', jnp.asarray(s1),
                     params['conv_W1'][:, 0, :]).reshape(H, P1 * 8)
    m2f = jnp.einsum('qpk,ck->pcq', jnp.asarray(s2),
                     params['conv_W2'][0]).reshape(P1 * 8, TW)
    c1b = jnp.tile(params['conv_b1'], P1)[None, :]
    b2row = jnp.broadcast_to(params['conv_b2'], (TW,))[None, :]
    return _tc_decoder(h, m1f, c1b, m2f, b2row, u)


# speedup vs baseline: 2.0492x; 1.0464x over previous
import functools

import jax
import jax.numpy as jnp
import numpy as np
from jax import lax
from jax.experimental import pallas as pl
from jax.experimental.pallas import tpu as pltpu
from jax.experimental.pallas import tpu_sc as plsc

N = 10000
E = 320000
TW = 25
H = 128
L_PDE = 16.0
TMAX = 4.0
DT = 0.01
EPS = 1e-5

NC = 2
NS = 16
NW = NC * NS
CHUNK = 80
EPAD = E
EPW = EPAD // NW
CPW = EPW // CHUNK
NPAD = 10240
NPT = NPAD // NS


_f32 = jnp.float32


def _swish(x):
    return x * (1.0 / (1.0 + jnp.exp(-x)))


def _tc_embed(ucat, w1, b1, w2, b2):
    blk = 1000

    def body(u_ref, w1_ref, b1_ref, w2_ref, b2_ref, h_ref):
        x = jnp.dot(u_ref[...], w1_ref[...], preferred_element_type=_f32)
        x = _swish(x + b1_ref[...])
        y = jnp.dot(x, w2_ref[...], preferred_element_type=_f32)
        h_ref[...] = _swish(y + b2_ref[...])

    return pl.pallas_call(
        body,
        grid=(N // blk,),
        in_specs=[
            pl.BlockSpec((blk, 32), lambda i: (i, 0)),
            pl.BlockSpec((32, H), lambda i: (0, 0)),
            pl.BlockSpec((1, H), lambda i: (0, 0)),
            pl.BlockSpec((H, H), lambda i: (0, 0)),
            pl.BlockSpec((1, H), lambda i: (0, 0)),
        ],
        out_specs=pl.BlockSpec((blk, H), lambda i: (i, 0)),
        out_shape=jax.ShapeDtypeStruct((N, H), _f32),
    )(ucat, w1, b1, w2, b2)


def _tc_pre(h, ucat, wa, wb, wcd, wcs, m1b, ua, ucu, u1b):
    blk = 1000

    def body(h_ref, u_ref, wa_ref, wb_ref, wcd_ref, wcs_ref, m1b_ref,
             ua_ref, ucu_ref, u1b_ref, a_ref, b_ref, p_ref):
        hv = h_ref[...]
        uv = u_ref[...]
        a_ref[...] = (jnp.dot(hv, wa_ref[...], preferred_element_type=_f32)
                      + jnp.dot(uv, wcd_ref[...], preferred_element_type=_f32)
                      + m1b_ref[...])
        b_ref[...] = (jnp.dot(hv, wb_ref[...], preferred_element_type=_f32)
                      + jnp.dot(uv, wcs_ref[...], preferred_element_type=_f32))
        p_ref[...] = (jnp.dot(hv, ua_ref[...], preferred_element_type=_f32)
                      + jnp.dot(uv, ucu_ref[...], preferred_element_type=_f32)
                      + u1b_ref[...])

    wspec = lambda shape: pl.BlockSpec(shape, lambda i: (0, 0))
    return pl.pallas_call(
        body,
        grid=(N // blk,),
        in_specs=[
            pl.BlockSpec((blk, H), lambda i: (i, 0)),
            pl.BlockSpec((blk, 32), lambda i: (i, 0)),
            wspec((H, H)), wspec((H, H)), wspec((32, H)), wspec((32, H)),
            wspec((1, H)), wspec((H, H)), wspec((32, H)), wspec((1, H)),
        ],
        out_specs=[pl.BlockSpec((blk, H), lambda i: (i, 0))] * 3,
        out_shape=[jax.ShapeDtypeStruct((N, H), _f32)] * 3,
    )(h, ucat, wa, wb, wcd, wcs, m1b, ua, ucu, u1b)


def _tc_edge(ad, bs, m2w, m2b):
    blk = 2000

    def body(a_ref, b_ref, w_ref, bias_ref, m_ref):
        t = _swish(a_ref[...] + b_ref[...])
        y = jnp.dot(t, w_ref[...], preferred_element_type=_f32) + bias_ref[...]
        m_ref[...] = _swish(y)

    return pl.pallas_call(
        body,
        grid=(EPAD // blk,),
        in_specs=[
            pl.BlockSpec((blk, H), lambda i: (i, 0)),
            pl.BlockSpec((blk, H), lambda i: (i, 0)),
            pl.BlockSpec((H, H), lambda i: (0, 0)),
            pl.BlockSpec((1, H), lambda i: (0, 0)),
        ],
        out_specs=pl.BlockSpec((blk, H), lambda i: (i, 0)),
        out_shape=jax.ShapeDtypeStruct((EPAD, H), _f32),
    )(ad, bs, m2w, m2b)


def _tc_upd(h, preu, p_parts, q_parts, degp, ub, u2w, u2b):
    def body(h_ref, pre_ref, p_ref, q_ref, dg_ref, ub_ref, u2w_ref, u2b_ref,
             o_ref):
        deg = jnp.maximum(dg_ref[0, :N] + dg_ref[1, :N], 1.0)
        agg = ((p_ref[0, :N] + p_ref[1, :N] + q_ref[0, :N] + q_ref[1, :N])
               * (1.0 / deg))
        x = pre_ref[...] + jnp.dot(agg, ub_ref[...], preferred_element_type=_f32)
        x = _swish(x)
        y = jnp.dot(x, u2w_ref[...], preferred_element_type=_f32) + u2b_ref[...]
        hn = h_ref[...] + _swish(y)
        mean = jnp.mean(hn, axis=0, keepdims=True)
        d = hn - mean
        var = jnp.mean(d * d, axis=0, keepdims=True)
        o_ref[...] = d / jnp.sqrt(var + EPS)

    return pl.pallas_call(
        body,
        out_shape=jax.ShapeDtypeStruct((N, H), _f32),
    )(h, preu, p_parts, q_parts, degp, ub, u2w, u2b)


def _tc_decoder(h, m1f, c1b, m2f, b2row, u):
    blk = 1000

    def body(h_ref, m1_ref, c1_ref, m2_ref, b2_ref, u_ref, o_ref):
        z = jnp.dot(h_ref[...], m1_ref[...], preferred_element_type=_f32)
        z = _swish(z + c1_ref[...])
        diff = jnp.dot(z, m2_ref[...], preferred_element_type=_f32) + b2_ref[...]
        dtv = DT * (lax.broadcasted_iota(jnp.int32, (blk, TW), 1)
                    .astype(_f32) + 1.0)
        o_ref[...] = u_ref[:, TW - 1:TW] + dtv * diff

    return pl.pallas_call(
        body,
        grid=(N // blk,),
        in_specs=[
            pl.BlockSpec((blk, H), lambda i: (i, 0)),
            pl.BlockSpec((H, 304), lambda i: (0, 0)),
            pl.BlockSpec((1, 304), lambda i: (0, 0)),
            pl.BlockSpec((304, TW), lambda i: (0, 0)),
            pl.BlockSpec((1, TW), lambda i: (0, 0)),
            pl.BlockSpec((blk, TW), lambda i: (i, 0)),
        ],
        out_specs=pl.BlockSpec((blk, TW), lambda i: (i, 0)),
        out_shape=jax.ShapeDtypeStruct((N, TW), _f32),
    )(h, m1f, c1b, m2f, b2row, u)


def _sc_gather_add(a, b, dstg3, srcg3, ne=EPAD, chunk=CHUNK, cpw=CPW):
    mesh = plsc.VectorSubcoreMesh(core_axis_name="c", subcore_axis_name="s")
    epw = ne // NW
    CHUNK_, CPW_ = chunk, cpw

    @functools.partial(
        pl.kernel,
        mesh=mesh,
        out_type=jax.ShapeDtypeStruct((ne, H), _f32),
        scratch_types=[
            pltpu.VMEM((CPW_, CHUNK_), jnp.int32),
            pltpu.VMEM((CPW_, CHUNK_), jnp.int32),
            pltpu.VMEM((4, CHUNK_, H), _f32),
            pltpu.SemaphoreType.DMA,
            pltpu.SemaphoreType.DMA,
            pltpu.SemaphoreType.DMA,
            pltpu.SemaphoreType.DMA,
        ],
    )
    def k(a_h, b_h, d3_h, s3_h, e_h, idx_d, idx_s, buf, s0, s1, s2, s3):
        c = lax.axis_index("c")
        s = lax.axis_index("s")
        w = s * NC + c
        pltpu.sync_copy(d3_h.at[w], idx_d)
        pltpu.sync_copy(s3_h.at[w], idx_s)
        base = w * epw
        sems = (s0, s1, s2, s3)
        CHUNK, CPW = CHUNK_, CPW_

        def issue_a(g, slot):
            pltpu.async_copy(a_h.at[idx_d.at[g]], buf.at[slot], sems[slot])

        def issue_b(g, slot):
            pltpu.async_copy(b_h.at[idx_s.at[g]], buf.at[slot], sems[slot],
                             add=True)

        def wait_buf(slot):
            pltpu.make_async_copy(a_h.at[idx_d.at[0]], buf.at[slot],
                                  sems[slot]).wait()

        def step(cur, j):
            wait_buf(j)
            pltpu.sync_copy(buf.at[j],
                            e_h.at[pl.ds(base + cur * CHUNK, CHUNK)])

            @pl.when(cur + 4 < CPW)
            def _():
                issue_a(cur + 4, j)

            nslot = (j + 2) % 4

            @pl.when(cur + 2 < CPW)
            def _():
                wait_buf(nslot)
                issue_b(cur + 2, nslot)

        for j in range(4):
            issue_a(j, j)
        for j in range(2):
            wait_buf(j)
            issue_b(j, j)

        @pl.loop(0, CPW - 1, step=4)
        def _(g):
            for j in range(4):
                step(g + j, j)

        step(CPW - 1, (CPW - 1) % 4)

    return k(a, b, dstg3, srcg3)


def _tc_edge_fused(e, m2w, m2b, ne=EPAD):
    blk = 2000

    def body(e_ref, w_ref, bias_ref, m_ref):
        t = _swish(e_ref[...])
        y = jnp.dot(t, w_ref[...], preferred_element_type=_f32) + bias_ref[...]
        m_ref[...] = _swish(y)

    return pl.pallas_call(
        body,
        grid=(ne // blk,),
        in_specs=[
            pl.BlockSpec((blk, H), lambda i: (i, 0)),
            pl.BlockSpec((H, H), lambda i: (0, 0)),
            pl.BlockSpec((1, H), lambda i: (0, 0)),
        ],
        out_specs=pl.BlockSpec((blk, H), lambda i: (i, 0)),
        out_shape=jax.ShapeDtypeStruct((ne, H), _f32),
    )(e, m2w, m2b)


def _sc_gather(a, b, dstg3, srcg3):
    mesh = plsc.VectorSubcoreMesh(core_axis_name="c", subcore_axis_name="s")

    @functools.partial(
        pl.kernel,
        mesh=mesh,
        out_type=[jax.ShapeDtypeStruct((EPAD, H), _f32)] * 2,
        scratch_types=[
            pltpu.VMEM((CPW, CHUNK), jnp.int32),
            pltpu.VMEM((CPW, CHUNK), jnp.int32),
            pltpu.VMEM((2, CHUNK, H), _f32),
            pltpu.VMEM((2, CHUNK, H), _f32),
            pltpu.SemaphoreType.DMA,
            pltpu.SemaphoreType.DMA,
        ],
    )
    def k(a_h, b_h, d3_h, s3_h, ad_h, bs_h, idx_d, idx_s, buf_a, buf_b, s0, s1):
        c = lax.axis_index("c")
        s = lax.axis_index("s")
        w = s * NC + c
        pltpu.sync_copy(d3_h.at[w], idx_d)
        pltpu.sync_copy(s3_h.at[w], idx_s)
        base = w * EPW
        sems = (s0, s1)

        def issue(g, slot):
            pltpu.async_copy(a_h.at[idx_d.at[g]], buf_a.at[slot], sems[slot])
            pltpu.async_copy(b_h.at[idx_s.at[g]], buf_b.at[slot], sems[slot])

        def drain(slot, cur):
            pltpu.make_async_copy(a_h.at[idx_d.at[0]], buf_a.at[slot],
                                  sems[slot]).wait()
            pltpu.make_async_copy(b_h.at[idx_s.at[0]], buf_b.at[slot],
                                  sems[slot]).wait()
            pltpu.sync_copy(buf_a.at[slot],
                            ad_h.at[pl.ds(base + cur * CHUNK, CHUNK)])
            pltpu.sync_copy(buf_b.at[slot],
                            bs_h.at[pl.ds(base + cur * CHUNK, CHUNK)])

        issue(0, 0)
        issue(1, 1)

        @pl.loop(0, CPW - 1, step=2)
        def _(g):
            for bslot in range(2):
                cur = g + bslot
                drain(bslot, cur)

                @pl.when(cur + 2 < CPW)
                def _():
                    issue(cur + 2, bslot)

        drain((CPW - 1) % 2, CPW - 1)

    return k(a, b, dstg3, srcg3)


def _sc_scatter(m, dst3, zeros_nd, ne=EPAD, chunk=CHUNK, cpw=CPW):
    mesh = plsc.VectorSubcoreMesh(core_axis_name="c", subcore_axis_name="s")
    epw = ne // NW
    CHUNK_, CPW_ = chunk, cpw

    @functools.partial(
        pl.kernel,
        mesh=mesh,
        out_type=jax.ShapeDtypeStruct((2, NPAD, H), _f32),
        scratch_types=[
            pltpu.VMEM((CPW_, CHUNK_), jnp.int32),
            pltpu.VMEM((2, CHUNK_, H), _f32),
            pltpu.VMEM_SHARED((NPAD, H), _f32),
            pltpu.SemaphoreType.DMA,
            pltpu.SemaphoreType.DMA,
        ],
    )
    def k(m_h, d3_h, z_h, out_h, idx_d, mbuf, acc, s0, s1):
        CHUNK, CPW = CHUNK_, CPW_
        c = lax.axis_index("c")
        s = lax.axis_index("s")
        w = s * NC + c
        pltpu.sync_copy(d3_h.at[w], idx_d)
        pltpu.sync_copy(z_h.at[pl.ds(s * NPT, NPT)], acc.at[pl.ds(s * NPT, NPT)])
        plsc.subcore_barrier()
        base = w * epw
        sems = (s0, s1)

        def load(g, slot):
            pltpu.async_copy(m_h.at[pl.ds(base + g * CHUNK, CHUNK)],
                             mbuf.at[slot], sems[slot])

        def drain(slot, cur):
            pltpu.make_async_copy(m_h.at[pl.ds(0, CHUNK)],
                                  mbuf.at[slot], sems[slot]).wait()
            pltpu.sync_copy(mbuf.at[slot], acc.at[idx_d.at[cur]], add=True)

        load(0, 0)
        load(1, 1)

        @pl.loop(0, CPW - 1, step=2)
        def _(g):
            for bslot in range(2):
                cur = g + bslot
                drain(bslot, cur)

                @pl.when(cur + 2 < CPW)
                def _():
                    load(cur + 2, bslot)

        drain((CPW - 1) % 2, CPW - 1)

        plsc.subcore_barrier()
        pltpu.sync_copy(acc.at[pl.ds(s * NPT, NPT)],
                        out_h.at[c, pl.ds(s * NPT, NPT)])

    return k(m, dst3, zeros_nd)


def _sc_degree(dst3, zeros_nw, ones_cw):
    mesh = plsc.VectorSubcoreMesh(core_axis_name="c", subcore_axis_name="s")
    DW = 128

    @functools.partial(
        pl.kernel,
        mesh=mesh,
        out_type=jax.ShapeDtypeStruct((2, NPAD, DW), _f32),
        scratch_types=[
            pltpu.VMEM((CPW, CHUNK), jnp.int32),
            pltpu.VMEM((CHUNK, DW), _f32),
            pltpu.VMEM_SHARED((NPAD, DW), _f32),
        ],
    )
    def k(d3_h, z_h, o_h, out_h, idx_d, ones_v, acc):
        c = lax.axis_index("c")
        s = lax.axis_index("s")
        w = s * NC + c
        pltpu.sync_copy(d3_h.at[w], idx_d)
        pltpu.sync_copy(o_h, ones_v)
        pltpu.sync_copy(z_h.at[pl.ds(s * NPT, NPT)], acc.at[pl.ds(s * NPT, NPT)])
        plsc.subcore_barrier()

        @pl.loop(0, CPW)
        def _(g):
            pltpu.sync_copy(ones_v, acc.at[idx_d.at[g]], add=True)

        plsc.subcore_barrier()
        pltpu.sync_copy(acc.at[pl.ds(s * NPT, NPT)],
                        out_h.at[c, pl.ds(s * NPT, NPT)])

    return k(dst3, zeros_nw, ones_cw)


def kernel(u, pos, edge_index, batch, params):
    del batch
    pos_x = pos[:, 1:2] / L_PDE
    variables = pos[:, 0:1] / TMAX
    ucat = jnp.concatenate(
        [u, pos_x, variables, jnp.zeros((N, 5), _f32)], axis=1)

    w1p = jnp.concatenate(
        [params['emb_W1'], jnp.zeros((5, H), _f32)], axis=0)
    h = _tc_embed(ucat, w1p, params['emb_b1'][None, :],
                  params['emb_W2'], params['emb_b2'][None, :])

    src = edge_index[0]
    dst = edge_index[1]
    dstg3 = dst.reshape(NW, CPW, CHUNK)
    dsts3 = dstg3
    EH = E // 2
    CH2 = 40
    CPW2 = (EH // NW) // CH2
    d3a = dst[:EH].reshape(NW, CPW2, CH2)
    d3b = dst[EH:].reshape(NW, CPW2, CH2)
    s3a = src[:EH].reshape(NW, CPW2, CH2)
    s3b = src[EH:].reshape(NW, CPW2, CH2)

    zeros_nd = jnp.zeros((NPAD, H), _f32)
    zeros_nw = jnp.zeros((NPAD, 128), _f32)
    ones_cw = jnp.ones((CHUNK, 128), _f32)
    degp = _sc_degree(dsts3, zeros_nw, ones_cw)[:, :, 0:1]

    pad5 = jnp.zeros((5, H), _f32)
    pad1 = jnp.zeros((1, H), _f32)
    for lp in params['layers']:
        w = lp['m1W']
        wa = w[0:H]
        wb = w[H:2 * H]
        wc = w[2 * H:2 * H + TW]
        wd = w[2 * H + TW:2 * H + TW + 1]
        we = w[2 * H + TW + 1:2 * H + TW + 2]
        wcd = jnp.concatenate([wc, wd, we, pad5], axis=0)
        wcs = jnp.concatenate([-wc, -wd, pad1, pad5], axis=0)
        uw = lp['u1W']
        ua = uw[0:H]
        ub = uw[H:2 * H]
        uc = uw[2 * H:2 * H + 1]
        ucu = jnp.concatenate([jnp.zeros((TW + 1, H), _f32), uc, pad5], axis=0)

        a, b, preu = _tc_pre(h, ucat, wa, wb, wcd, wcs, lp['m1b'][None, :],
                             ua, ucu, lp['u1b'][None, :])
        e1 = _sc_gather_add(a, b, d3a, s3a, ne=EH, chunk=CH2, cpw=CPW2)
        m1 = _tc_edge_fused(e1, lp['m2W'], lp['m2b'][None, :], ne=EH)
        e2 = _sc_gather_add(a, b, d3b, s3b, ne=EH, chunk=CH2, cpw=CPW2)
        m2 = _tc_edge_fused(e2, lp['m2W'], lp['m2b'][None, :], ne=EH)
        p1 = _sc_scatter(m1, d3a, zeros_nd, ne=EH, chunk=CH2, cpw=CPW2)
        p2 = _sc_scatter(m2, d3b, zeros_nd, ne=EH, chunk=CH2, cpw=CPW2)
        h = _tc_upd(h, preu, p1, p2, degp, ub, lp['u2W'],
                    lp['u2b'][None, :])

    P1 = (H - 16) // 3 + 1
    s1 = np.zeros((P1, H, 16), np.float32)
    for p in range(P1):
        for kk in range(16):
            s1[p, 3 * p + kk, kk] = 1.0
    s2 = np.zeros((TW, P1, 14), np.float32)
    for q in range(TW):
        for kk in range(14):
            s2[q, q + kk, kk] = 1.0
    m1f = jnp.einsum('pjk,ck->jpc', jnp.asarray(s1),
                     params['conv_W1'][:, 0, :]).reshape(H, P1 * 8)
    m2f = jnp.einsum('qpk,ck->pcq', jnp.asarray(s2),
                     params['conv_W2'][0]).reshape(P1 * 8, TW)
    c1b = jnp.tile(params['conv_b1'], P1)[None, :]
    b2row = jnp.broadcast_to(params['conv_b2'], (TW,))[None, :]
    return _tc_decoder(h, m1f, c1b, m2f, b2row, u)
